# Initial kernel scaffold; baseline (speedup 1.0000x reference)
import jax
import jax.numpy as jnp
from jax.experimental import pallas as pl


def kernel(x, table):
    raise NotImplementedError("write your pallas kernel here")


# speedup vs baseline: 1.3067x; 1.3067x over previous
import functools

import jax
import jax.numpy as jnp
from jax import lax
from jax.experimental import pallas as pl
from jax.experimental.pallas import tpu as pltpu
from jax.experimental.pallas import tpu_sc as plsc

NUM_WORKERS = 32
GROUP = 128


@jax.jit
def kernel(x, table):
    B, S = x.shape
    V, D = table.shape
    N = B * S
    n_per_w = N // NUM_WORKERS
    n_groups = n_per_w // GROUP

    idx = x.reshape(NUM_WORKERS, n_groups, GROUP).astype(jnp.int32)

    mesh = plsc.VectorSubcoreMesh(core_axis_name="c", subcore_axis_name="s")

    @functools.partial(
        pl.kernel,
        mesh=mesh,
        out_type=jax.ShapeDtypeStruct((N, D), jnp.float32),
        compiler_params=pltpu.CompilerParams(use_tc_tiling_on_sc=False),
        scratch_types=[
            pltpu.VMEM((n_groups, GROUP), jnp.int32),
            pltpu.VMEM((GROUP, D), jnp.float32),
            pltpu.SemaphoreType.DMA,
        ],
    )
    def emb(table_hbm, idx_hbm, out_hbm, idx_v, buf, sem):
        wid = lax.axis_index("s") * 2 + lax.axis_index("c")
        base = wid * n_per_w
        pltpu.sync_copy(idx_hbm.at[wid], idx_v)

        def body(g, carry):
            pltpu.async_copy(table_hbm.at[idx_v.at[g]], buf, sem).wait()
            pltpu.sync_copy(buf, out_hbm.at[pl.ds(base + g * GROUP, GROUP)])
            return carry

        lax.fori_loop(0, n_groups, body, 0)

    out = emb(table, idx)
    return out.reshape(B, S, D)


# speedup vs baseline: 1.4918x; 1.1417x over previous
import functools

import jax
import jax.numpy as jnp
from jax import lax
from jax.experimental import pallas as pl
from jax.experimental.pallas import tpu as pltpu
from jax.experimental.pallas import tpu_sc as plsc

NUM_WORKERS = 32
GROUP = 128
K = 8


@jax.jit
def kernel(x, table):
    B, S = x.shape
    V, D = table.shape
    N = B * S
    n_per_w = N // NUM_WORKERS
    n_groups = n_per_w // GROUP
    n_blocks = n_groups // K

    idx = x.reshape(NUM_WORKERS, n_groups, GROUP).astype(jnp.int32)

    mesh = plsc.VectorSubcoreMesh(core_axis_name="c", subcore_axis_name="s")

    @functools.partial(
        pl.kernel,
        mesh=mesh,
        out_type=jax.ShapeDtypeStruct((N, D), jnp.float32),
        compiler_params=pltpu.CompilerParams(use_tc_tiling_on_sc=False),
        scratch_types=[
            pltpu.VMEM((n_groups, GROUP), jnp.int32),
            pltpu.VMEM((2 * K, GROUP, D), jnp.float32),
            pltpu.SemaphoreType.DMA,
            pltpu.SemaphoreType.DMA,
        ],
    )
    def emb(table_hbm, idx_hbm, out_hbm, idx_v, buf, gsem, wsem):
        wid = lax.axis_index("s") * 2 + lax.axis_index("c")
        base = wid * n_per_w
        pltpu.sync_copy(idx_hbm.at[wid], idx_v)

        for b in range(K):
            pltpu.async_copy(table_hbm.at[idx_v.at[b]], buf.at[b], gsem)

        def body(blk, carry):
            bank = (blk % 2) * K
            obank = ((blk + 1) % 2) * K

            for b in range(K):
                pltpu.make_async_copy(
                    table_hbm.at[idx_v.at[blk * K + b]], buf.at[bank + b], gsem
                ).wait()

            @pl.when(blk >= 1)
            def _():
                for b in range(K):
                    g = (blk - 1) * K + b
                    pltpu.make_async_copy(
                        buf.at[obank + b],
                        out_hbm.at[pl.ds(base + g * GROUP, GROUP)],
                        wsem,
                    ).wait()

            @pl.when(blk + 1 < n_blocks)
            def _():
                for b in range(K):
                    g = (blk + 1) * K + b
                    pltpu.async_copy(
                        table_hbm.at[idx_v.at[g]], buf.at[obank + b], gsem
                    )

            for b in range(K):
                g = blk * K + b
                pltpu.async_copy(
                    buf.at[bank + b],
                    out_hbm.at[pl.ds(base + g * GROUP, GROUP)],
                    wsem,
                )
            return carry

        lax.fori_loop(0, n_blocks, body, 0)

        last_bank = ((n_blocks - 1) % 2) * K
        for b in range(K):
            g = (n_blocks - 1) * K + b
            pltpu.make_async_copy(
                buf.at[last_bank + b],
                out_hbm.at[pl.ds(base + g * GROUP, GROUP)],
                wsem,
            ).wait()

    out = emb(table, idx)
    return out.reshape(B, S, D)


# speedup vs baseline: 1.4937x; 1.0013x over previous
import functools

import jax
import jax.numpy as jnp
from jax import lax
from jax.experimental import pallas as pl
from jax.experimental.pallas import tpu as pltpu
from jax.experimental.pallas import tpu_sc as plsc

NUM_WORKERS = 32
GROUP = 1024
K = 1


@jax.jit
def kernel(x, table):
    B, S = x.shape
    V, D = table.shape
    N = B * S
    n_per_w = N // NUM_WORKERS
    n_groups = n_per_w // GROUP
    n_blocks = n_groups // K

    idx = x.reshape(NUM_WORKERS, n_groups, GROUP).astype(jnp.int32)

    mesh = plsc.VectorSubcoreMesh(core_axis_name="c", subcore_axis_name="s")

    @functools.partial(
        pl.kernel,
        mesh=mesh,
        out_type=jax.ShapeDtypeStruct((N, D), jnp.float32),
        compiler_params=pltpu.CompilerParams(use_tc_tiling_on_sc=False),
        scratch_types=[
            pltpu.VMEM((n_groups, GROUP), jnp.int32),
            pltpu.VMEM((2 * K, GROUP, D), jnp.float32),
            pltpu.SemaphoreType.DMA,
            pltpu.SemaphoreType.DMA,
        ],
    )
    def emb(table_hbm, idx_hbm, out_hbm, idx_v, buf, gsem, wsem):
        wid = lax.axis_index("s") * 2 + lax.axis_index("c")
        base = wid * n_per_w
        pltpu.sync_copy(idx_hbm.at[wid], idx_v)

        for b in range(K):
            pltpu.async_copy(table_hbm.at[idx_v.at[b]], buf.at[b], gsem)

        def body(blk, carry):
            bank = (blk % 2) * K
            obank = ((blk + 1) % 2) * K

            for b in range(K):
                pltpu.make_async_copy(
                    table_hbm.at[idx_v.at[blk * K + b]], buf.at[bank + b], gsem
                ).wait()

            @pl.when(blk >= 1)
            def _():
                for b in range(K):
                    g = (blk - 1) * K + b
                    pltpu.make_async_copy(
                        buf.at[obank + b],
                        out_hbm.at[pl.ds(base + g * GROUP, GROUP)],
                        wsem,
                    ).wait()

            @pl.when(blk + 1 < n_blocks)
            def _():
                for b in range(K):
                    g = (blk + 1) * K + b
                    pltpu.async_copy(
                        table_hbm.at[idx_v.at[g]], buf.at[obank + b], gsem
                    )

            for b in range(K):
                g = blk * K + b
                pltpu.async_copy(
                    buf.at[bank + b],
                    out_hbm.at[pl.ds(base + g * GROUP, GROUP)],
                    wsem,
                )
            return carry

        lax.fori_loop(0, n_blocks, body, 0)

        last_bank = ((n_blocks - 1) % 2) * K
        for b in range(K):
            g = (n_blocks - 1) * K + b
            pltpu.make_async_copy(
                buf.at[last_bank + b],
                out_hbm.at[pl.ds(base + g * GROUP, GROUP)],
                wsem,
            ).wait()

    out = emb(table, idx)
    return out.reshape(B, S, D)
